# baseline (speedup 1.0000x reference)
import functools

import jax
import jax.numpy as jnp
from jax import lax
from jax.experimental import pallas as pl
from jax.experimental.pallas import tpu as pltpu
from jax.experimental.pallas import tpu_sc as plsc

V = 1_000_000
D = 64
DP = 128
NC, NS = 2, 16
NW = NC * NS
TCOL = 128
NFULL = V // TCOL
TAIL = V - NFULL * TCOL
TN = NFULL // NW
TNB = 4
CHUNK = 128
NBUF = 4
NCB = 2
K = 2


UNROLL = 8


def _transpose_tile(a_v, b_v, ncols):
    lanes = [lax.iota(jnp.int32, 16) + 16 * k for k in range(D // 16)]
    zero = jnp.zeros((16,), jnp.int32)

    def col(i8, carry):
        base = zero + i8 * UNROLL
        for u in range(UNROLL):
            cols = base + u
            for k in range(D // 16):
                vals = plsc.load_gather(a_v, [lanes[k], cols])
                b_v[i8 * UNROLL + u, pl.ds(16 * k, 16)] = vals
        return carry

    lax.fori_loop(0, ncols // UNROLL, col, 0)


def _t_body(wt_hbm, wtail_hbm, t2_hbm, a_v, b_v, at_v, isem, osem):
    wid = lax.axis_index("s") * NC + lax.axis_index("c")

    def cof(j):
        return pl.multiple_of((wid + NW * j) * TCOL, TCOL)

    def ifire(j, slot):
        pltpu.async_copy(wt_hbm.at[:, pl.ds(cof(j), TCOL)], a_v.at[slot],
                         isem.at[slot])

    def iwait(j, slot):
        pltpu.make_async_copy(wt_hbm.at[:, pl.ds(cof(j), TCOL)],
                              a_v.at[slot], isem.at[slot]).wait()

    def ofire(j, slot):
        pltpu.async_copy(b_v.at[slot], t2_hbm.at[pl.ds(cof(j), TCOL)],
                         osem.at[slot])

    def owait(j, slot):
        pltpu.make_async_copy(b_v.at[slot], t2_hbm.at[pl.ds(cof(j), TCOL)],
                              osem.at[slot]).wait()

    def work(j, slot):
        _transpose_tile(a_v.at[slot], b_v.at[slot], TCOL)

    for b in range(K):
        ifire(b, b)
    for b in range(TNB):
        iwait(b, b)
        work(b, b)
        ofire(b, b)
        ifire(b + K, (b + K) % TNB)

    def group(g, carry):
        j0 = g * TNB
        for b in range(TNB):
            j = j0 + b
            iwait(j, b)
            owait(j - TNB, b)
            work(j, b)
            ofire(j, b)
            f = j + K
            c2 = wid + NW * f

            @pl.when(c2 < NFULL)
            def _():
                ifire(f, (b + K) % TNB)

        return carry

    lax.fori_loop(1, TN // TNB, group, 0)

    @pl.when(wid + NW * TN < NFULL)
    def _():
        s = TN % TNB
        iwait(TN, s)
        owait(TN - TNB, s)
        work(TN, s)
        ofire(TN, s)

    for b in range(TNB):
        pltpu.make_async_copy(b_v.at[b], t2_hbm.at[pl.ds(0, TCOL)],
                              osem.at[b]).wait()

    @pl.when(wid == 0)
    def _():
        pltpu.sync_copy(wtail_hbm, at_v)
        _transpose_tile(at_v, b_v.at[0], TAIL)
        pltpu.sync_copy(b_v.at[0, pl.ds(0, TAIL)],
                        t2_hbm.at[pl.ds(NFULL * TCOL, TAIL)])


def _g_body(idx_hbm, t2_hbm, out_hbm, idx_v, rows_v, c_v, gsem, ssem):
    wid = lax.axis_index("s") * NC + lax.axis_index("c")
    nch = idx_v.shape[0]
    base = wid * (nch * CHUNK)
    pltpu.sync_copy(idx_hbm.at[wid], idx_v)

    def gfire(j, slot):
        pltpu.async_copy(t2_hbm.at[idx_v.at[j]], rows_v.at[slot],
                         gsem.at[slot])

    def gwait(j, slot):
        pltpu.make_async_copy(t2_hbm.at[idx_v.at[j]], rows_v.at[slot],
                              gsem.at[slot]).wait()

    def sfire(j, cs):
        pltpu.async_copy(c_v.at[cs],
                         out_hbm.at[pl.ds(base + j * CHUNK, CHUNK)],
                         ssem.at[cs])

    def swait(j, cs):
        pltpu.make_async_copy(c_v.at[cs],
                              out_hbm.at[pl.ds(base + j * CHUNK, CHUNK)],
                              ssem.at[cs]).wait()

    def compact(slot, cs):
        def row(i8, carry):
            for u in range(UNROLL):
                i = i8 * UNROLL + u
                for k in range(D // 16):
                    c_v[cs, i, pl.ds(16 * k, 16)] = (
                        rows_v[slot, i, pl.ds(16 * k, 16)])
            return carry

        lax.fori_loop(0, CHUNK // UNROLL, row, 0)

    for b in range(K):
        gfire(b, b)

    for b in range(NBUF):
        gwait(b, b)
        if b >= NCB:
            swait(b - NCB, b % NCB)
        compact(b, b % NCB)
        sfire(b, b % NCB)
        gfire(b + K, (b + K) % NBUF)

    def group(g, carry):
        j0 = g * NBUF
        for b in range(NBUF):
            j = j0 + b
            cs = b % NCB
            gwait(j, b)
            swait(j - NCB, cs)
            compact(b, cs)
            sfire(j, cs)

            @pl.when(j + K < nch)
            def _():
                gfire(j + K, (b + K) % NBUF)

        return carry

    lax.fori_loop(1, nch // NBUF, group, 0)

    for m in range(NCB):
        jf = nch - NCB + m
        swait(jf, jf % NCB)


@functools.partial(jax.jit)
def kernel(token_ids, weight):
    bt, lt = token_ids.shape
    n = bt * lt
    nch = n // (NW * CHUNK)
    idx = token_ids.reshape(NW, nch, CHUNK).astype(jnp.int32)
    mesh = plsc.VectorSubcoreMesh(core_axis_name="c", subcore_axis_name="s")
    params = pltpu.CompilerParams(needs_layout_passes=False)

    prep = pl.kernel(
        _t_body,
        mesh=mesh,
        out_type=jax.ShapeDtypeStruct((V, DP), jnp.float32),
        scratch_types=[
            pltpu.VMEM((TNB, D, TCOL), jnp.float32),
            pltpu.VMEM((TNB, TCOL, DP), jnp.float32),
            pltpu.VMEM((D, TAIL), jnp.float32),
            pltpu.SemaphoreType.DMA((TNB,)),
            pltpu.SemaphoreType.DMA((TNB,)),
        ],
        compiler_params=params,
    )
    lookup = pl.kernel(
        _g_body,
        mesh=mesh,
        out_type=jax.ShapeDtypeStruct((n, D), jnp.float32),
        scratch_types=[
            pltpu.VMEM((nch, CHUNK), jnp.int32),
            pltpu.VMEM((NBUF, CHUNK, DP), jnp.float32),
            pltpu.VMEM((NCB, CHUNK, D), jnp.float32),
            pltpu.SemaphoreType.DMA((NBUF,)),
            pltpu.SemaphoreType.DMA((NCB,)),
        ],
        compiler_params=params,
    )
    t2 = prep(weight.T, weight[NFULL * TCOL:, :].T)
    out = lookup(idx, t2)
    return out.reshape(bt, lt, D)


# speedup vs baseline: 1.5568x; 1.5568x over previous
import functools

import jax
import jax.numpy as jnp
from jax import lax
from jax.experimental import pallas as pl
from jax.experimental.pallas import tpu as pltpu
from jax.experimental.pallas import tpu_sc as plsc

V = 1_000_000
D = 64
DP = 128
NC, NS = 2, 16
NW = NC * NS
TCOL = 128
NFULL = V // TCOL
TAIL = V - NFULL * TCOL
TN = NFULL // NW
TNB = 4
CHUNK = 128
NBUF = 4
NCB = 2
K = 2


UNROLL = 8


def _transpose_tile(a_v, b_v, ncols):
    lanes = [lax.iota(jnp.int32, 16) + 16 * k for k in range(D // 16)]
    zero = jnp.zeros((16,), jnp.int32)

    @plsc.parallel_loop(0, ncols, unroll=UNROLL)
    def _(i):
        cols = zero + i
        for k in range(D // 16):
            vals = plsc.load_gather(a_v, [lanes[k], cols])
            b_v[i, pl.ds(16 * k, 16)] = vals


def _t_body(wt_hbm, wtail_hbm, t2_hbm, a_v, b_v, at_v, isem, osem):
    wid = lax.axis_index("s") * NC + lax.axis_index("c")

    def cof(j):
        return pl.multiple_of((wid + NW * j) * TCOL, TCOL)

    def ifire(j, slot):
        pltpu.async_copy(wt_hbm.at[:, pl.ds(cof(j), TCOL)], a_v.at[slot],
                         isem.at[slot])

    def iwait(j, slot):
        pltpu.make_async_copy(wt_hbm.at[:, pl.ds(cof(j), TCOL)],
                              a_v.at[slot], isem.at[slot]).wait()

    def ofire(j, slot):
        pltpu.async_copy(b_v.at[slot], t2_hbm.at[pl.ds(cof(j), TCOL)],
                         osem.at[slot])

    def owait(j, slot):
        pltpu.make_async_copy(b_v.at[slot], t2_hbm.at[pl.ds(cof(j), TCOL)],
                              osem.at[slot]).wait()

    def work(j, slot):
        _transpose_tile(a_v.at[slot], b_v.at[slot], TCOL)

    for b in range(K):
        ifire(b, b)
    for b in range(TNB):
        iwait(b, b)
        work(b, b)
        ofire(b, b)
        ifire(b + K, (b + K) % TNB)

    def group(g, carry):
        j0 = g * TNB
        for b in range(TNB):
            j = j0 + b
            iwait(j, b)
            owait(j - TNB, b)
            work(j, b)
            ofire(j, b)
            f = j + K
            c2 = wid + NW * f

            @pl.when(c2 < NFULL)
            def _():
                ifire(f, (b + K) % TNB)

        return carry

    lax.fori_loop(1, TN // TNB, group, 0)

    @pl.when(wid + NW * TN < NFULL)
    def _():
        s = TN % TNB
        iwait(TN, s)
        owait(TN - TNB, s)
        work(TN, s)
        ofire(TN, s)

    for b in range(TNB):
        pltpu.make_async_copy(b_v.at[b], t2_hbm.at[pl.ds(0, TCOL)],
                              osem.at[b]).wait()

    @pl.when(wid == 0)
    def _():
        pltpu.sync_copy(wtail_hbm, at_v)
        _transpose_tile(at_v, b_v.at[0], TAIL)
        pltpu.sync_copy(b_v.at[0, pl.ds(0, TAIL)],
                        t2_hbm.at[pl.ds(NFULL * TCOL, TAIL)])


def _g_body(idx_hbm, t2_hbm, out_hbm, idx_v, rows_v, c_v, gsem, ssem):
    wid = lax.axis_index("s") * NC + lax.axis_index("c")
    nch = idx_v.shape[0]
    base = wid * (nch * CHUNK)
    pltpu.sync_copy(idx_hbm.at[wid], idx_v)

    def gfire(j, slot):
        pltpu.async_copy(t2_hbm.at[idx_v.at[j]], rows_v.at[slot],
                         gsem.at[slot])

    def gwait(j, slot):
        pltpu.make_async_copy(t2_hbm.at[idx_v.at[j]], rows_v.at[slot],
                              gsem.at[slot]).wait()

    def sfire(j, cs):
        pltpu.async_copy(c_v.at[cs],
                         out_hbm.at[pl.ds(base + j * CHUNK, CHUNK)],
                         ssem.at[cs])

    def swait(j, cs):
        pltpu.make_async_copy(c_v.at[cs],
                              out_hbm.at[pl.ds(base + j * CHUNK, CHUNK)],
                              ssem.at[cs]).wait()

    def compact(slot, cs):
        @plsc.parallel_loop(0, CHUNK, unroll=UNROLL)
        def _(i):
            for k in range(D // 16):
                c_v[cs, i, pl.ds(16 * k, 16)] = (
                    rows_v[slot, i, pl.ds(16 * k, 16)])

    for b in range(K):
        gfire(b, b)

    for b in range(NBUF):
        gwait(b, b)
        if b >= NCB:
            swait(b - NCB, b % NCB)
        compact(b, b % NCB)
        sfire(b, b % NCB)
        gfire(b + K, (b + K) % NBUF)

    def group(g, carry):
        j0 = g * NBUF
        for b in range(NBUF):
            j = j0 + b
            cs = b % NCB
            gwait(j, b)
            swait(j - NCB, cs)
            compact(b, cs)
            sfire(j, cs)

            @pl.when(j + K < nch)
            def _():
                gfire(j + K, (b + K) % NBUF)

        return carry

    lax.fori_loop(1, nch // NBUF, group, 0)

    for m in range(NCB):
        jf = nch - NCB + m
        swait(jf, jf % NCB)


@functools.partial(jax.jit)
def kernel(token_ids, weight):
    bt, lt = token_ids.shape
    n = bt * lt
    nch = n // (NW * CHUNK)
    idx = token_ids.reshape(NW, nch, CHUNK).astype(jnp.int32)
    mesh = plsc.VectorSubcoreMesh(core_axis_name="c", subcore_axis_name="s")
    params = pltpu.CompilerParams(needs_layout_passes=False)

    prep = pl.kernel(
        _t_body,
        mesh=mesh,
        out_type=jax.ShapeDtypeStruct((V, DP), jnp.float32),
        scratch_types=[
            pltpu.VMEM((TNB, D, TCOL), jnp.float32),
            pltpu.VMEM((TNB, TCOL, DP), jnp.float32),
            pltpu.VMEM((D, TAIL), jnp.float32),
            pltpu.SemaphoreType.DMA((TNB,)),
            pltpu.SemaphoreType.DMA((TNB,)),
        ],
        compiler_params=params,
    )
    lookup = pl.kernel(
        _g_body,
        mesh=mesh,
        out_type=jax.ShapeDtypeStruct((n, D), jnp.float32),
        scratch_types=[
            pltpu.VMEM((nch, CHUNK), jnp.int32),
            pltpu.VMEM((NBUF, CHUNK, DP), jnp.float32),
            pltpu.VMEM((NCB, CHUNK, D), jnp.float32),
            pltpu.SemaphoreType.DMA((NBUF,)),
            pltpu.SemaphoreType.DMA((NCB,)),
        ],
        compiler_params=params,
    )
    t2 = prep(weight.T, weight[NFULL * TCOL:, :].T)
    out = lookup(idx, t2)
    return out.reshape(bt, lt, D)


# speedup vs baseline: 1.6067x; 1.0321x over previous
import functools

import jax
import jax.numpy as jnp
from jax import lax
from jax.experimental import pallas as pl
from jax.experimental.pallas import tpu as pltpu
from jax.experimental.pallas import tpu_sc as plsc

D = 64
NC, NS = 2, 16
NW = NC * NS
CHUNK = 128
NBUF = 8
K = 4


def _emb_body(idx_hbm, table_hbm, out_hbm, idx_v, rows_v, gsem, ssem):
    wid = lax.axis_index("s") * NC + lax.axis_index("c")
    nch = idx_v.shape[0]
    ngrp = nch // NBUF
    base = wid * (nch * CHUNK)
    pltpu.sync_copy(idx_hbm.at[wid], idx_v)

    def gfire(j, slot):
        pltpu.async_copy(table_hbm.at[idx_v.at[j]], rows_v.at[slot],
                         gsem.at[slot])

    def gwait(j, slot):
        pltpu.make_async_copy(table_hbm.at[idx_v.at[j]], rows_v.at[slot],
                              gsem.at[slot]).wait()

    def sfire(j, slot):
        pltpu.async_copy(rows_v.at[slot],
                         out_hbm.at[pl.ds(base + j * CHUNK, CHUNK)],
                         ssem.at[slot])

    def swait(j, slot):
        pltpu.make_async_copy(rows_v.at[slot],
                              out_hbm.at[pl.ds(base + j * CHUNK, CHUNK)],
                              ssem.at[slot]).wait()

    for b in range(K):
        gfire(b, b)

    for b in range(NBUF):
        gwait(b, b)
        sfire(b, b)
        f = b + K
        if f < NBUF:
            gfire(f, f)
        else:
            swait(f - NBUF, f - NBUF)
            gfire(f, f - NBUF)

    def group(g, carry):
        i0 = g * NBUF
        for b in range(NBUF):
            i = i0 + b
            s = (b + K) % NBUF
            gwait(i, b)
            sfire(i, b)
            swait(i + K - NBUF, s)
            gfire(i + K, s)
        return carry

    lax.fori_loop(1, ngrp - 1, group, 0)

    i0 = (ngrp - 1) * NBUF
    for b in range(NBUF):
        i = i0 + b
        gwait(i, b)
        sfire(i, b)
        if b + K < NBUF:
            s = (b + K) % NBUF
            swait(i + K - NBUF, s)
            gfire(i + K, s)

    for b in range(NBUF):
        swait(i0 + b, b)


@functools.partial(jax.jit)
def kernel(token_ids, weight):
    bt, lt = token_ids.shape
    n = bt * lt
    nch = n // (NW * CHUNK)
    idx = token_ids.reshape(NW, nch, CHUNK).astype(jnp.int32)
    mesh = plsc.VectorSubcoreMesh(core_axis_name="c", subcore_axis_name="s")
    run = pl.kernel(
        _emb_body,
        mesh=mesh,
        out_type=jax.ShapeDtypeStruct((n, D), jnp.float32),
        scratch_types=[
            pltpu.VMEM((nch, CHUNK), jnp.int32),
            pltpu.VMEM((NBUF, CHUNK, D), jnp.float32),
            pltpu.SemaphoreType.DMA((NBUF,)),
            pltpu.SemaphoreType.DMA((NBUF,)),
        ],
        compiler_params=pltpu.CompilerParams(use_tc_tiling_on_sc=False),
    )
    out = run(idx, weight)
    return out.reshape(bt, lt, D)
